# baseline (speedup 1.0000x reference)
import jax
import jax.numpy as jnp
from jax.experimental import pallas as pl
from jax.experimental.pallas import tpu as pltpu

_B = 64
_C = 4
_HW = 224 * 224
_BB = 4
_NPIX = _B * _HW


def _loss_kernel(x_ref, y_ref, out_ref, acc_ref):
    step = pl.program_id(0)

    @pl.when(step == 0)
    def _init():
        acc_ref[0] = 0.0
        acc_ref[1] = 0.0
        acc_ref[2] = 0.0

    p = x_ref[:, 0, :]
    t = y_ref[:, 0, :]
    logp = jnp.maximum(jnp.log(p), -100.0)
    log1mp = jnp.maximum(jnp.log(1.0 - p), -100.0)
    bce = -(t * logp + (1.0 - t) * log1mp)
    pt = jnp.exp(-bce)
    one_m_pt = 1.0 - pt
    f = 0.9 * one_m_pt * one_m_pt * bce

    mask = t != 0.0
    cnt = jnp.sum(mask.astype(jnp.float32))

    d = y_ref[:, 1:, :] - x_ref[:, 1:, :]
    sq = d * d
    msq = jnp.sum(jnp.where(mask[:, None, :], sq, 0.0))

    acc_ref[0] += jnp.sum(f)
    acc_ref[1] += msq
    acc_ref[2] += cnt

    @pl.when(step == pl.num_programs(0) - 1)
    def _fini():
        out_ref[0] = acc_ref[0] / _NPIX + acc_ref[1] / (acc_ref[2] * 3.0)


def kernel(inputs, target):
    x = inputs.reshape(_B, _C, _HW)
    y = target.reshape(_B, _C, _HW)
    spec = pl.BlockSpec((_BB, _C, _HW), lambda b: (b, 0, 0))
    out = pl.pallas_call(
        _loss_kernel,
        grid=(_B // _BB,),
        in_specs=[spec, spec],
        out_specs=pl.BlockSpec(memory_space=pltpu.SMEM),
        out_shape=jax.ShapeDtypeStruct((1,), jnp.float32),
        scratch_shapes=[pltpu.SMEM((3,), jnp.float32)],
    )(x, y)
    return out[0]


# speedup vs baseline: 4.9009x; 4.9009x over previous
import jax
import jax.numpy as jnp
from jax.experimental import pallas as pl
from jax.experimental.pallas import tpu as pltpu

_B = 64
_C = 4
_H = 224
_W = 224
_BB = 4
_NPIX = _B * _H * _W


def _loss_kernel(x_ref, y_ref, out_ref, acc_ref):
    step = pl.program_id(0)

    @pl.when(step == 0)
    def _init():
        acc_ref[0] = 0.0
        acc_ref[1] = 0.0
        acc_ref[2] = 0.0

    p = x_ref[:, 0]
    t = y_ref[:, 0]
    logp = jnp.maximum(jnp.log(p), -100.0)
    log1mp = jnp.maximum(jnp.log(1.0 - p), -100.0)
    bce = -(t * logp + (1.0 - t) * log1mp)
    pt = jnp.exp(-bce)
    one_m_pt = 1.0 - pt
    f = 0.9 * one_m_pt * one_m_pt * bce

    mask = t != 0.0
    cnt = jnp.sum(mask.astype(jnp.float32))

    d = y_ref[:, 1:] - x_ref[:, 1:]
    sq = d * d
    msq = jnp.sum(jnp.where(mask[:, None], sq, 0.0))

    acc_ref[0] += jnp.sum(f)
    acc_ref[1] += msq
    acc_ref[2] += cnt

    @pl.when(step == pl.num_programs(0) - 1)
    def _fini():
        out_ref[0] = acc_ref[0] / _NPIX + acc_ref[1] / (acc_ref[2] * 3.0)


def kernel(inputs, target):
    spec = pl.BlockSpec((_BB, _C, _H, _W), lambda b: (b, 0, 0, 0))
    out = pl.pallas_call(
        _loss_kernel,
        grid=(_B // _BB,),
        in_specs=[spec, spec],
        out_specs=pl.BlockSpec(memory_space=pltpu.SMEM),
        out_shape=jax.ShapeDtypeStruct((1,), jnp.float32),
        scratch_shapes=[pltpu.SMEM((3,), jnp.float32)],
    )(inputs, target)
    return out[0]


# speedup vs baseline: 5.2458x; 1.0704x over previous
import jax
import jax.numpy as jnp
from jax.experimental import pallas as pl
from jax.experimental.pallas import tpu as pltpu

_B = 64
_C = 4
_H = 224
_W = 224
_BB = 8
_NPIX = _B * _H * _W


def _loss_kernel(x_ref, y_ref, out_ref, acc_ref):
    step = pl.program_id(0)

    @pl.when(step == 0)
    def _init():
        acc_ref[0] = 0.0
        acc_ref[1] = 0.0
        acc_ref[2] = 0.0

    p = x_ref[:, 0]
    t = y_ref[:, 0]
    logp = jnp.maximum(jnp.log(p), -100.0)
    log1mp = jnp.maximum(jnp.log(1.0 - p), -100.0)
    bce = -(t * logp + (1.0 - t) * log1mp)
    pt = jnp.exp(-bce)
    one_m_pt = 1.0 - pt
    f = 0.9 * one_m_pt * one_m_pt * bce

    mask = t != 0.0
    cnt = jnp.sum(mask.astype(jnp.float32))

    d = y_ref[:, 1:] - x_ref[:, 1:]
    sq = d * d
    msq = jnp.sum(jnp.where(mask[:, None], sq, 0.0))

    acc_ref[0] += jnp.sum(f)
    acc_ref[1] += msq
    acc_ref[2] += cnt

    @pl.when(step == pl.num_programs(0) - 1)
    def _fini():
        out_ref[0] = acc_ref[0] / _NPIX + acc_ref[1] / (acc_ref[2] * 3.0)


def kernel(inputs, target):
    spec = pl.BlockSpec((_BB, _C, _H, _W), lambda b: (b, 0, 0, 0))
    out = pl.pallas_call(
        _loss_kernel,
        grid=(_B // _BB,),
        in_specs=[spec, spec],
        out_specs=pl.BlockSpec(memory_space=pltpu.SMEM),
        out_shape=jax.ShapeDtypeStruct((1,), jnp.float32),
        scratch_shapes=[pltpu.SMEM((3,), jnp.float32)],
    )(inputs, target)
    return out[0]
